# Initial kernel scaffold; baseline (speedup 1.0000x reference)
import jax
import jax.numpy as jnp
from jax.experimental import pallas as pl


def kernel(x, edge_index, W_l1, W_r1, b1, W_l2, W_r2, b2, W_l3, W_r3, b3, gn1, bt1, gn2, bt2, gn3, bt3, Wres1, bres1, Wres2, bres2, Wres3, bres3, Wc1, bc1, Wc2, bc2, We1, bwe1, We2, bwe2):
    raise NotImplementedError("write your pallas kernel here")


# speedup vs baseline: 4.4680x; 4.4680x over previous
import functools

import jax
import jax.numpy as jnp
from jax import lax
from jax.experimental import pallas as pl
from jax.experimental.pallas import tpu as pltpu
from jax.experimental.pallas import tpu_sc as plsc

NN = 50000
NC, NS, LANES = 2, 16, 16
NW = NC * NS
STRIPE = 3128
N_ACC = NS * STRIPE
E_PAD = 819200
IDX_ROWS = E_PAD // 128
ROWS_PER_TILE = IDX_ROWS // NW
CHUNK_ROWS = 8
NCHUNK = ROWS_PER_TILE // CHUNK_ROWS
RB = 1000


def _make_agg(nb):
  mesh = plsc.VectorSubcoreMesh(core_axis_name="c", subcore_axis_name="s",
                                num_cores=NC, num_subcores=NS)

  @functools.partial(
      pl.kernel,
      out_type=jax.ShapeDtypeStruct((NC, nb, N_ACC, LANES), jnp.float32),
      mesh=mesh,
      scratch_types=[
          pltpu.VMEM_SHARED((N_ACC, LANES), jnp.float32),
          pltpu.VMEM((CHUNK_ROWS, 128), jnp.int32),
          pltpu.VMEM((CHUNK_ROWS, 128), jnp.int32),
          pltpu.VMEM((CHUNK_ROWS * 128, LANES), jnp.float32),
          pltpu.VMEM((STRIPE, LANES), jnp.float32),
          pltpu.SemaphoreType.DMA,
      ],
      compiler_params=pltpu.CompilerParams(use_tc_tiling_on_sc=False),
  )
  def agg(feat_hbm, src_hbm, dst_hbm, out_hbm, acc, sbuf, dbuf, gbuf, zbuf,
          gsem):
    c = lax.axis_index("c")
    s = lax.axis_index("s")

    def zb(i, carry):
      zbuf[i, :] = jnp.zeros((LANES,), jnp.float32)
      return carry

    lax.fori_loop(0, STRIPE, zb, 0)

    row0 = (c * NS + s) * ROWS_PER_TILE

    for b in range(nb):
      pltpu.sync_copy(zbuf, acc.at[pl.ds(s * STRIPE, STRIPE)])
      plsc.subcore_barrier()

      def chunk_body(t, carry):
        r = row0 + t * CHUNK_ROWS
        pltpu.sync_copy(src_hbm.at[pl.ds(r, CHUNK_ROWS)], sbuf)
        pltpu.sync_copy(dst_hbm.at[pl.ds(r, CHUNK_ROWS)], dbuf)
        descs = []
        for j in range(CHUNK_ROWS):
          descs.append(
              pltpu.async_copy(feat_hbm.at[b].at[sbuf.at[j]],
                               gbuf.at[pl.ds(j * 128, 128)], gsem))
        for d in descs:
          d.wait()
        for j in range(CHUNK_ROWS):
          pltpu.sync_copy(gbuf.at[pl.ds(j * 128, 128)], acc.at[dbuf.at[j]],
                          add=True)
        return carry

      lax.fori_loop(0, NCHUNK, chunk_body, 0)
      plsc.subcore_barrier()
      pltpu.sync_copy(acc.at[pl.ds(s * STRIPE, STRIPE)],
                      out_hbm.at[c, b].at[pl.ds(s * STRIPE, STRIPE)])

  return agg


_agg_cache = {}


def _agg(nb, feat, src_p, dst_p):
  if nb not in _agg_cache:
    _agg_cache[nb] = _make_agg(nb)
  return _agg_cache[nb](feat, src_p, dst_p)


def _full(shape):
  return pl.BlockSpec(shape, lambda i: tuple(0 for _ in shape))


def _rows(shape):
  nd = len(shape)
  if nd == 4:
    return pl.BlockSpec(shape, lambda i: (0, 0, i, 0))
  if nd == 3:
    return pl.BlockSpec(shape, lambda i: (0, i, 0))
  return pl.BlockSpec(shape, lambda i: (i, 0))


def _dense1(s1, f1, wl1, wr1, wres1, b1f, bres1):
  def body(s1_ref, f1_ref, wl1_ref, wr1_ref, wres1_ref, b1_ref, bres1_ref,
           h1_ref, rcnt_ref):
    ssum = s1_ref[0, 0] + s1_ref[1, 0]
    cnt = ssum[:, 10:11]
    rc = 1.0 / jnp.clip(cnt, 1.0, None)
    mean = ssum * rc
    x = f1_ref[...]
    h = (jnp.dot(mean, wl1_ref[...], preferred_element_type=jnp.float32)
         + jnp.dot(x, wr1_ref[...], preferred_element_type=jnp.float32)
         + b1_ref[...])
    h = jnp.maximum(h, 0.0) + jnp.dot(
        x, wres1_ref[...], preferred_element_type=jnp.float32) + bres1_ref[...]
    rcnt_ref[...] = jnp.broadcast_to(rc, (RB, 8))
    for b in range(4):
      h1_ref[b] = h[:, b * 16:(b + 1) * 16]

  return pl.pallas_call(
      body,
      grid=(NN // RB,),
      in_specs=[
          _rows((2, 1, RB, 16)),
          _rows((RB, 16)),
          _full((16, 64)),
          _full((16, 64)),
          _full((16, 64)),
          _full((1, 64)),
          _full((1, 64)),
      ],
      out_specs=[_rows((4, RB, 16)), _rows((RB, 8))],
      out_shape=[
          jax.ShapeDtypeStruct((4, NN, 16), jnp.float32),
          jax.ShapeDtypeStruct((NN, 8), jnp.float32),
      ],
  )(s1, f1, wl1, wr1, wres1, b1f, bres1)


def _dense2(s2, h1b, rcnt, wl2, wr2, wres2, wl3, b2f, bres2):
  def body(s2_ref, h1_ref, rc_ref, wl2_ref, wr2_ref, wres2_ref, wl3_ref,
           b2_ref, bres2_ref, h2_ref, g3_ref):
    rc = rc_ref[:, 0:1]
    ssum = s2_ref[0] + s2_ref[1]
    h1 = jnp.concatenate([h1_ref[b] for b in range(4)], axis=1)
    acc = jnp.dot(h1, wr2_ref[...], preferred_element_type=jnp.float32)
    for b in range(4):
      mean_b = ssum[b] * rc
      acc = acc + jnp.dot(mean_b, wl2_ref[pl.ds(b * 16, 16), :],
                          preferred_element_type=jnp.float32)
    h = acc + b2_ref[...]
    h2 = jnp.maximum(h, 0.0) + jnp.dot(
        h1, wres2_ref[...], preferred_element_type=jnp.float32) + bres2_ref[...]
    h2_ref[...] = h2
    g3 = jnp.dot(h2, wl3_ref[...], preferred_element_type=jnp.float32)
    for b in range(4):
      g3_ref[b] = g3[:, b * 16:(b + 1) * 16]

  return pl.pallas_call(
      body,
      grid=(NN // RB,),
      in_specs=[
          _rows((2, 4, RB, 16)),
          _rows((4, RB, 16)),
          _rows((RB, 8)),
          _full((64, 128)),
          _full((64, 128)),
          _full((64, 128)),
          _full((128, 64)),
          _full((1, 128)),
          _full((1, 128)),
      ],
      out_specs=[_rows((RB, 128)), _rows((4, RB, 16))],
      out_shape=[
          jax.ShapeDtypeStruct((NN, 128), jnp.float32),
          jax.ShapeDtypeStruct((4, NN, 16), jnp.float32),
      ],
  )(s2, h1b, rcnt, wl2, wr2, wres2, wl3, b2f, bres2)


def _dense3(s3, h2, rcnt, wr3, wres3, b3f, bres3, wc1, bc1, wc2p, bc2p, we1,
            bwe1, we2p, bwe2p):
  def body(s3_ref, h2_ref, rc_ref, wr3_ref, wres3_ref, b3_ref, bres3_ref,
           wc1_ref, bc1_ref, wc2_ref, bc2_ref, we1_ref, bwe1_ref, we2_ref,
           bwe2_ref, h3_ref, mule_ref, risk_ref):
    rc = rc_ref[:, 0:1]
    mean = jnp.concatenate([s3_ref[0, b] + s3_ref[1, b] for b in range(4)],
                           axis=1) * rc
    h2 = h2_ref[...]
    h = mean + jnp.dot(h2, wr3_ref[...],
                       preferred_element_type=jnp.float32) + b3_ref[...]
    h3 = jnp.maximum(h, 0.0) + jnp.dot(
        h2, wres3_ref[...], preferred_element_type=jnp.float32) + bres3_ref[...]
    h3_ref[...] = h3
    c1 = jnp.maximum(
        jnp.dot(h3, wc1_ref[...], preferred_element_type=jnp.float32)
        + bc1_ref[...], 0.0)
    logits = jnp.dot(c1, wc2_ref[...],
                     preferred_element_type=jnp.float32) + bc2_ref[...]
    mule_ref[...] = jax.nn.sigmoid(logits)
    e1 = jnp.maximum(
        jnp.dot(h3, we1_ref[...], preferred_element_type=jnp.float32)
        + bwe1_ref[...], 0.0)
    el = jnp.dot(e1, we2_ref[...],
                 preferred_element_type=jnp.float32) + bwe2_ref[...]
    el = el - jnp.max(el, axis=1, keepdims=True)
    ex = jnp.exp(el)
    risk_ref[...] = ex / jnp.sum(ex, axis=1, keepdims=True)

  return pl.pallas_call(
      body,
      grid=(NN // RB,),
      in_specs=[
          _rows((2, 4, RB, 16)),
          _rows((RB, 128)),
          _rows((RB, 8)),
          _full((128, 64)),
          _full((128, 64)),
          _full((1, 64)),
          _full((1, 64)),
          _full((64, 32)),
          _full((1, 32)),
          _full((32, 8)),
          _full((1, 8)),
          _full((64, 16)),
          _full((1, 16)),
          _full((16, 8)),
          _full((1, 8)),
      ],
      out_specs=[_rows((RB, 64)), _rows((RB, 8)), _rows((RB, 8))],
      out_shape=[
          jax.ShapeDtypeStruct((NN, 64), jnp.float32),
          jax.ShapeDtypeStruct((NN, 8), jnp.float32),
          jax.ShapeDtypeStruct((NN, 8), jnp.float32),
      ],
  )(s3, h2, rcnt, wr3, wres3, b3f, bres3, wc1, bc1, wc2p, bc2p, we1, bwe1,
    we2p, bwe2p)


def kernel(x, edge_index, W_l1, W_r1, b1, W_l2, W_r2, b2, W_l3, W_r3, b3,
           gn1, bt1, gn2, bt2, gn3, bt3,
           Wres1, bres1, Wres2, bres2, Wres3, bres3,
           Wc1, bc1, Wc2, bc2, We1, bwe1, We2, bwe2):
  n, in_c = x.shape
  e = edge_index.shape[1]

  eps = 1e-5
  sc1 = gn1 / jnp.sqrt(1.0 + eps)
  sc2 = gn2 / jnp.sqrt(1.0 + eps)
  sc3 = gn3 / jnp.sqrt(1.0 + eps)

  def pad_rows(w, rows):
    return jnp.concatenate(
        [w, jnp.zeros((rows - w.shape[0], w.shape[1]), w.dtype)], axis=0)

  wl1 = pad_rows(W_l1 * sc1[None, :], 16)
  wr1 = pad_rows(W_r1 * sc1[None, :], 16)
  wres1 = pad_rows(Wres1, 16)
  b1f = (b1 * sc1 + bt1)[None, :]
  bres1f = bres1[None, :]

  wl2 = W_l2 * sc2[None, :]
  wr2 = W_r2 * sc2[None, :]
  b2f = (b2 * sc2 + bt2)[None, :]
  bres2f = bres2[None, :]

  wl3s = W_l3 * sc3[None, :]
  wr3 = W_r3 * sc3[None, :]
  b3f = (b3 * sc3 + bt3)[None, :]
  bres3f = bres3[None, :]

  wc2p = jnp.concatenate([Wc2, jnp.zeros((32, 7), jnp.float32)], axis=1)
  bc2p = jnp.concatenate([bc2, jnp.zeros((7,), jnp.float32)])[None, :]
  we2p = jnp.concatenate([We2, jnp.zeros((16, 3), jnp.float32)], axis=1)
  bwe2p = jnp.concatenate(
      [bwe2, jnp.full((3,), -1e30, jnp.float32)])[None, :]
  bc1f = bc1[None, :]
  bwe1f = bwe1[None, :]

  src = edge_index[0]
  dst = edge_index[1]
  src_p = jnp.concatenate(
      [src, jnp.zeros((E_PAD - e,), jnp.int32)]).reshape(IDX_ROWS, 128)
  dst_p = jnp.concatenate(
      [dst, jnp.full((E_PAD - e,), NN, jnp.int32)]).reshape(IDX_ROWS, 128)

  f1 = jnp.concatenate(
      [x, jnp.ones((n, 1), jnp.float32), jnp.zeros((n, 5), jnp.float32)],
      axis=1)

  s1 = _agg(1, f1.reshape(1, n, 16), src_p, dst_p)
  h1b, rcnt = _dense1(s1, f1, wl1, wr1, wres1, b1f, bres1f)
  s2 = _agg(4, h1b, src_p, dst_p)
  h2, g3b = _dense2(s2, h1b, rcnt, wl2, wr2, Wres2, wl3s, b2f, bres2f)
  s3 = _agg(4, g3b, src_p, dst_p)
  h3, mule, risk = _dense3(s3, h2, rcnt, wr3, Wres3, b3f, bres3f, Wc1, bc1f,
                           wc2p, bc2p, We1, bwe1f, we2p, bwe2p)

  return (mule[:, 0], h3, risk[:, :5])


# speedup vs baseline: 5.8535x; 1.3101x over previous
import functools

import jax
import jax.numpy as jnp
from jax import lax
from jax.experimental import pallas as pl
from jax.experimental.pallas import tpu as pltpu
from jax.experimental.pallas import tpu_sc as plsc

NN = 50000
NC, NS, LANES = 2, 16, 16
NW = NC * NS
STRIPE = 3128
N_ACC = NS * STRIPE
E_PAD = 819200
IDX_ROWS = E_PAD // 128
ROWS_PER_TILE = IDX_ROWS // NW
IDR = 10
DEPTH = 5
NOUTER = ROWS_PER_TILE // IDR
ZROWS = 64
RB = 1000


def _make_agg(nb, w):
  mesh = plsc.VectorSubcoreMesh(core_axis_name="c", subcore_axis_name="s",
                                num_cores=NC, num_subcores=NS)

  @functools.partial(
      pl.kernel,
      out_type=jax.ShapeDtypeStruct((NC, nb, N_ACC, w), jnp.float32),
      mesh=mesh,
      scratch_types=(
          [
              pltpu.VMEM_SHARED((N_ACC, w), jnp.float32),
              pltpu.VMEM((IDR, 128), jnp.int32),
              pltpu.VMEM((IDR, 128), jnp.int32),
              pltpu.VMEM((ZROWS, w), jnp.float32),
          ]
          + [pltpu.VMEM((128, w), jnp.float32) for _ in range(DEPTH)]
          + [pltpu.SemaphoreType.DMA for _ in range(2 * DEPTH)]
      ),
      compiler_params=pltpu.CompilerParams(use_tc_tiling_on_sc=False),
  )
  def agg(feat_hbm, src_hbm, dst_hbm, out_hbm, acc, sbuf, dbuf, zbuf, *rest):
    gbufs = rest[:DEPTH]
    gsems = rest[DEPTH:2 * DEPTH]
    ssems = rest[2 * DEPTH:]
    c = lax.axis_index("c")
    s = lax.axis_index("s")

    def zb(i, carry):
      for q in range(w // LANES):
        zbuf[i, pl.ds(q * LANES, LANES)] = jnp.zeros((LANES,), jnp.float32)
      return carry

    lax.fori_loop(0, ZROWS, zb, 0)

    row0 = (c * NS + s) * ROWS_PER_TILE

    for b in range(nb):
      for k in range(STRIPE // ZROWS):
        pltpu.sync_copy(zbuf, acc.at[pl.ds(s * STRIPE + k * ZROWS, ZROWS)])
      rem = STRIPE % ZROWS
      if rem:
        pltpu.sync_copy(
            zbuf.at[pl.ds(0, rem)],
            acc.at[pl.ds(s * STRIPE + (STRIPE // ZROWS) * ZROWS, rem)])
      plsc.subcore_barrier()

      def iter_body(t, carry):
        r = row0 + t * IDR
        pltpu.sync_copy(src_hbm.at[pl.ds(r, IDR)], sbuf)
        pltpu.sync_copy(dst_hbm.at[pl.ds(r, IDR)], dbuf)
        g = [None] * IDR
        sdesc = [None] * IDR
        for j in range(DEPTH):
          g[j] = pltpu.async_copy(feat_hbm.at[b].at[sbuf.at[j]], gbufs[j],
                                  gsems[j])
        for j in range(IDR):
          d = j % DEPTH
          g[j].wait()
          sdesc[j] = pltpu.async_copy(gbufs[d], acc.at[dbuf.at[j]], ssems[d],
                                      add=True)
          if j + DEPTH < IDR:
            sdesc[j].wait()
            g[j + DEPTH] = pltpu.async_copy(
                feat_hbm.at[b].at[sbuf.at[j + DEPTH]], gbufs[d], gsems[d])
        for j in range(IDR - DEPTH, IDR):
          sdesc[j].wait()
        return carry

      lax.fori_loop(0, NOUTER, iter_body, 0)
      plsc.subcore_barrier()
      pltpu.sync_copy(acc.at[pl.ds(s * STRIPE, STRIPE)],
                      out_hbm.at[c, b].at[pl.ds(s * STRIPE, STRIPE)])

  return agg


_agg_cache = {}


def _agg(nb, w, feat, src_p, dst_p):
  if (nb, w) not in _agg_cache:
    _agg_cache[(nb, w)] = _make_agg(nb, w)
  return _agg_cache[(nb, w)](feat, src_p, dst_p)


def _full(shape):
  return pl.BlockSpec(shape, lambda i: tuple(0 for _ in shape))


def _rows(shape):
  nd = len(shape)
  if nd == 4:
    return pl.BlockSpec(shape, lambda i: (0, 0, i, 0))
  if nd == 3:
    return pl.BlockSpec(shape, lambda i: (0, i, 0))
  return pl.BlockSpec(shape, lambda i: (i, 0))


def _dense1(s1, f1, wl1, wr1, wres1, b1f, bres1):
  def body(s1_ref, f1_ref, wl1_ref, wr1_ref, wres1_ref, b1_ref, bres1_ref,
           h1_ref, rcnt_ref):
    ssum = s1_ref[0, 0] + s1_ref[1, 0]
    cnt = ssum[:, 10:11]
    rc = 1.0 / jnp.clip(cnt, 1.0, None)
    mean = ssum * rc
    x = f1_ref[...]
    h = (jnp.dot(mean, wl1_ref[...], preferred_element_type=jnp.float32)
         + jnp.dot(x, wr1_ref[...], preferred_element_type=jnp.float32)
         + b1_ref[...])
    h = jnp.maximum(h, 0.0) + jnp.dot(
        x, wres1_ref[...], preferred_element_type=jnp.float32) + bres1_ref[...]
    rcnt_ref[...] = jnp.broadcast_to(rc, (RB, 8))
    for b in range(2):
      h1_ref[b] = h[:, b * 32:(b + 1) * 32]

  return pl.pallas_call(
      body,
      grid=(NN // RB,),
      in_specs=[
          _rows((2, 1, RB, 16)),
          _rows((RB, 16)),
          _full((16, 64)),
          _full((16, 64)),
          _full((16, 64)),
          _full((1, 64)),
          _full((1, 64)),
      ],
      out_specs=[_rows((2, RB, 32)), _rows((RB, 8))],
      out_shape=[
          jax.ShapeDtypeStruct((2, NN, 32), jnp.float32),
          jax.ShapeDtypeStruct((NN, 8), jnp.float32),
      ],
  )(s1, f1, wl1, wr1, wres1, b1f, bres1)


def _dense2(s2, h1b, rcnt, wl2, wr2, wres2, wl3, b2f, bres2):
  def body(s2_ref, h1_ref, rc_ref, wl2_ref, wr2_ref, wres2_ref, wl3_ref,
           b2_ref, bres2_ref, h2_ref, g3_ref):
    rc = rc_ref[:, 0:1]
    ssum = s2_ref[0] + s2_ref[1]
    h1 = jnp.concatenate([h1_ref[b] for b in range(2)], axis=1)
    acc = jnp.dot(h1, wr2_ref[...], preferred_element_type=jnp.float32)
    for b in range(2):
      mean_b = ssum[b] * rc
      acc = acc + jnp.dot(mean_b, wl2_ref[pl.ds(b * 32, 32), :],
                          preferred_element_type=jnp.float32)
    h = acc + b2_ref[...]
    h2 = jnp.maximum(h, 0.0) + jnp.dot(
        h1, wres2_ref[...], preferred_element_type=jnp.float32) + bres2_ref[...]
    h2_ref[...] = h2
    g3 = jnp.dot(h2, wl3_ref[...], preferred_element_type=jnp.float32)
    for b in range(2):
      g3_ref[b] = g3[:, b * 32:(b + 1) * 32]

  return pl.pallas_call(
      body,
      grid=(NN // RB,),
      in_specs=[
          _rows((2, 2, RB, 32)),
          _rows((2, RB, 32)),
          _rows((RB, 8)),
          _full((64, 128)),
          _full((64, 128)),
          _full((64, 128)),
          _full((128, 64)),
          _full((1, 128)),
          _full((1, 128)),
      ],
      out_specs=[_rows((RB, 128)), _rows((2, RB, 32))],
      out_shape=[
          jax.ShapeDtypeStruct((NN, 128), jnp.float32),
          jax.ShapeDtypeStruct((2, NN, 32), jnp.float32),
      ],
  )(s2, h1b, rcnt, wl2, wr2, wres2, wl3, b2f, bres2)


def _dense3(s3, h2, rcnt, wr3, wres3, b3f, bres3, wc1, bc1, wc2p, bc2p, we1,
            bwe1, we2p, bwe2p):
  def body(s3_ref, h2_ref, rc_ref, wr3_ref, wres3_ref, b3_ref, bres3_ref,
           wc1_ref, bc1_ref, wc2_ref, bc2_ref, we1_ref, bwe1_ref, we2_ref,
           bwe2_ref, h3_ref, mule_ref, risk_ref):
    rc = rc_ref[:, 0:1]
    mean = jnp.concatenate([s3_ref[0, b] + s3_ref[1, b] for b in range(2)],
                           axis=1) * rc
    h2 = h2_ref[...]
    h = mean + jnp.dot(h2, wr3_ref[...],
                       preferred_element_type=jnp.float32) + b3_ref[...]
    h3 = jnp.maximum(h, 0.0) + jnp.dot(
        h2, wres3_ref[...], preferred_element_type=jnp.float32) + bres3_ref[...]
    h3_ref[...] = h3
    c1 = jnp.maximum(
        jnp.dot(h3, wc1_ref[...], preferred_element_type=jnp.float32)
        + bc1_ref[...], 0.0)
    logits = jnp.dot(c1, wc2_ref[...],
                     preferred_element_type=jnp.float32) + bc2_ref[...]
    mule_ref[...] = jax.nn.sigmoid(logits)
    e1 = jnp.maximum(
        jnp.dot(h3, we1_ref[...], preferred_element_type=jnp.float32)
        + bwe1_ref[...], 0.0)
    el = jnp.dot(e1, we2_ref[...],
                 preferred_element_type=jnp.float32) + bwe2_ref[...]
    el = el - jnp.max(el, axis=1, keepdims=True)
    ex = jnp.exp(el)
    risk_ref[...] = ex / jnp.sum(ex, axis=1, keepdims=True)

  return pl.pallas_call(
      body,
      grid=(NN // RB,),
      in_specs=[
          _rows((2, 2, RB, 32)),
          _rows((RB, 128)),
          _rows((RB, 8)),
          _full((128, 64)),
          _full((128, 64)),
          _full((1, 64)),
          _full((1, 64)),
          _full((64, 32)),
          _full((1, 32)),
          _full((32, 8)),
          _full((1, 8)),
          _full((64, 16)),
          _full((1, 16)),
          _full((16, 8)),
          _full((1, 8)),
      ],
      out_specs=[_rows((RB, 64)), _rows((RB, 8)), _rows((RB, 8))],
      out_shape=[
          jax.ShapeDtypeStruct((NN, 64), jnp.float32),
          jax.ShapeDtypeStruct((NN, 8), jnp.float32),
          jax.ShapeDtypeStruct((NN, 8), jnp.float32),
      ],
  )(s3, h2, rcnt, wr3, wres3, b3f, bres3, wc1, bc1, wc2p, bc2p, we1, bwe1,
    we2p, bwe2p)


def kernel(x, edge_index, W_l1, W_r1, b1, W_l2, W_r2, b2, W_l3, W_r3, b3,
           gn1, bt1, gn2, bt2, gn3, bt3,
           Wres1, bres1, Wres2, bres2, Wres3, bres3,
           Wc1, bc1, Wc2, bc2, We1, bwe1, We2, bwe2):
  n, in_c = x.shape
  e = edge_index.shape[1]

  eps = 1e-5
  sc1 = gn1 / jnp.sqrt(1.0 + eps)
  sc2 = gn2 / jnp.sqrt(1.0 + eps)
  sc3 = gn3 / jnp.sqrt(1.0 + eps)

  def pad_rows(w, rows):
    return jnp.concatenate(
        [w, jnp.zeros((rows - w.shape[0], w.shape[1]), w.dtype)], axis=0)

  wl1 = pad_rows(W_l1 * sc1[None, :], 16)
  wr1 = pad_rows(W_r1 * sc1[None, :], 16)
  wres1 = pad_rows(Wres1, 16)
  b1f = (b1 * sc1 + bt1)[None, :]
  bres1f = bres1[None, :]

  wl2 = W_l2 * sc2[None, :]
  wr2 = W_r2 * sc2[None, :]
  b2f = (b2 * sc2 + bt2)[None, :]
  bres2f = bres2[None, :]

  wl3s = W_l3 * sc3[None, :]
  wr3 = W_r3 * sc3[None, :]
  b3f = (b3 * sc3 + bt3)[None, :]
  bres3f = bres3[None, :]

  wc2p = jnp.concatenate([Wc2, jnp.zeros((32, 7), jnp.float32)], axis=1)
  bc2p = jnp.concatenate([bc2, jnp.zeros((7,), jnp.float32)])[None, :]
  we2p = jnp.concatenate([We2, jnp.zeros((16, 3), jnp.float32)], axis=1)
  bwe2p = jnp.concatenate(
      [bwe2, jnp.full((3,), -1e30, jnp.float32)])[None, :]
  bc1f = bc1[None, :]
  bwe1f = bwe1[None, :]

  src = edge_index[0]
  dst = edge_index[1]
  npad = E_PAD - e
  pad_dst = NN + jax.lax.rem(jnp.arange(npad, dtype=jnp.int32),
                             jnp.int32(N_ACC - NN))
  src_p = jnp.concatenate(
      [src, jnp.zeros((npad,), jnp.int32)]).reshape(IDX_ROWS, 128)
  dst_p = jnp.concatenate([dst, pad_dst]).reshape(IDX_ROWS, 128)

  f1 = jnp.concatenate(
      [x, jnp.ones((n, 1), jnp.float32), jnp.zeros((n, 5), jnp.float32)],
      axis=1)

  s1 = _agg(1, 16, f1.reshape(1, n, 16), src_p, dst_p)
  h1b, rcnt = _dense1(s1, f1, wl1, wr1, wres1, b1f, bres1f)
  s2 = _agg(2, 32, h1b, src_p, dst_p)
  h2, g3b = _dense2(s2, h1b, rcnt, wl2, wr2, Wres2, wl3s, b2f, bres2f)
  s3 = _agg(2, 32, g3b, src_p, dst_p)
  h3, mule, risk = _dense3(s3, h2, rcnt, wr3, Wres3, b3f, bres3f, Wc1, bc1f,
                           wc2p, bc2p, We1, bwe1f, we2p, bwe2p)

  return (mule[:, 0], h3, risk[:, :5])


# speedup vs baseline: 8.4543x; 1.4443x over previous
import functools

import jax
import jax.numpy as jnp
from jax import lax
from jax.experimental import pallas as pl
from jax.experimental.pallas import tpu as pltpu
from jax.experimental.pallas import tpu_sc as plsc

NN = 50000
NC, NS, LANES = 2, 16, 16
NW = NC * NS
STRIPE = 3128
N_ACC = NS * STRIPE
E_PAD = 819200
IDX_ROWS = E_PAD // 128
ROWS_PER_TILE = IDX_ROWS // NW
IDR = 10
DEPTH = 5
NOUTER = ROWS_PER_TILE // IDR
ZROWS = 64
RB = 1000


def _make_agg(nb, w, dtype):
  mesh = plsc.VectorSubcoreMesh(core_axis_name="c", subcore_axis_name="s",
                                num_cores=NC, num_subcores=NS)

  @functools.partial(
      pl.kernel,
      out_type=jax.ShapeDtypeStruct((NC, nb, N_ACC, w), dtype),
      mesh=mesh,
      scratch_types=(
          [
              pltpu.VMEM_SHARED((N_ACC, w), dtype),
              pltpu.VMEM((IDR, 128), jnp.int32),
              pltpu.VMEM((IDR, 128), jnp.int32),
              pltpu.VMEM((ZROWS, w), dtype),
          ]
          + [pltpu.VMEM((128, w), dtype) for _ in range(DEPTH)]
          + [pltpu.SemaphoreType.DMA for _ in range(2 * DEPTH)]
      ),
      compiler_params=pltpu.CompilerParams(use_tc_tiling_on_sc=False),
  )
  def agg(feat_hbm, src_hbm, dst_hbm, out_hbm, acc, sbuf, dbuf, zbuf, *rest):
    gbufs = rest[:DEPTH]
    gsems = rest[DEPTH:2 * DEPTH]
    ssems = rest[2 * DEPTH:]
    c = lax.axis_index("c")
    s = lax.axis_index("s")

    def zb(i, carry):
      for q in range(w // LANES):
        zbuf[i, pl.ds(q * LANES, LANES)] = jnp.zeros((LANES,), dtype)
      return carry

    lax.fori_loop(0, ZROWS, zb, 0)

    row0 = (c * NS + s) * ROWS_PER_TILE

    for b in range(nb):
      for k in range(STRIPE // ZROWS):
        pltpu.sync_copy(zbuf, acc.at[pl.ds(s * STRIPE + k * ZROWS, ZROWS)])
      rem = STRIPE % ZROWS
      if rem:
        pltpu.sync_copy(
            zbuf.at[pl.ds(0, rem)],
            acc.at[pl.ds(s * STRIPE + (STRIPE // ZROWS) * ZROWS, rem)])
      plsc.subcore_barrier()

      def iter_body(t, carry):
        r = row0 + t * IDR
        pltpu.sync_copy(src_hbm.at[pl.ds(r, IDR)], sbuf)
        pltpu.sync_copy(dst_hbm.at[pl.ds(r, IDR)], dbuf)
        g = [None] * IDR
        sdesc = [None] * IDR
        for j in range(DEPTH):
          g[j] = pltpu.async_copy(feat_hbm.at[b].at[sbuf.at[j]], gbufs[j],
                                  gsems[j])
        for j in range(IDR):
          d = j % DEPTH
          g[j].wait()
          sdesc[j] = pltpu.async_copy(gbufs[d], acc.at[dbuf.at[j]], ssems[d],
                                      add=True)
          if j + DEPTH < IDR:
            sdesc[j].wait()
            g[j + DEPTH] = pltpu.async_copy(
                feat_hbm.at[b].at[sbuf.at[j + DEPTH]], gbufs[d], gsems[d])
        for j in range(IDR - DEPTH, IDR):
          sdesc[j].wait()
        return carry

      lax.fori_loop(0, NOUTER, iter_body, 0)
      plsc.subcore_barrier()
      pltpu.sync_copy(acc.at[pl.ds(s * STRIPE, STRIPE)],
                      out_hbm.at[c, b].at[pl.ds(s * STRIPE, STRIPE)])

  return agg


_agg_cache = {}


def _agg(nb, w, feat, src_p, dst_p):
  key = (nb, w, feat.dtype)
  if key not in _agg_cache:
    _agg_cache[key] = _make_agg(nb, w, feat.dtype)
  return _agg_cache[key](feat, src_p, dst_p)


def _full(shape):
  return pl.BlockSpec(shape, lambda i: tuple(0 for _ in shape))


def _rows(shape):
  nd = len(shape)
  if nd == 4:
    return pl.BlockSpec(shape, lambda i: (0, 0, i, 0))
  if nd == 3:
    return pl.BlockSpec(shape, lambda i: (0, i, 0))
  return pl.BlockSpec(shape, lambda i: (i, 0))


def _dense1(s1, f1, wl1, wr1, wres1, b1f, bres1):
  def body(s1_ref, f1_ref, wl1_ref, wr1_ref, wres1_ref, b1_ref, bres1_ref,
           h1_ref, rcnt_ref, st_ref):
    ssum = s1_ref[0, 0] + s1_ref[1, 0]
    cnt = ssum[:, 10:11]
    rc = 1.0 / jnp.clip(cnt, 1.0, None)
    mean = ssum * rc
    x = f1_ref[...]
    h = (jnp.dot(mean, wl1_ref[...], preferred_element_type=jnp.float32)
         + jnp.dot(x, wr1_ref[...], preferred_element_type=jnp.float32)
         + b1_ref[...])
    h = jnp.maximum(h, 0.0) + jnp.dot(
        x, wres1_ref[...], preferred_element_type=jnp.float32) + bres1_ref[...]
    rcnt_ref[...] = jnp.concatenate(
        [jnp.broadcast_to(rc, (RB, 4)), jnp.broadcast_to(cnt, (RB, 4))],
        axis=1)
    habs = jnp.max(jnp.abs(h), axis=0, keepdims=True)
    dmax = jnp.full((1, 64), jnp.max(cnt))
    st_ref[...] = jnp.broadcast_to(
        jnp.concatenate([habs, dmax], axis=1), (8, 128))
    for b in range(2):
      h1_ref[b] = h[:, b * 32:(b + 1) * 32]

  return pl.pallas_call(
      body,
      grid=(NN // RB,),
      in_specs=[
          _rows((2, 1, RB, 16)),
          _rows((RB, 16)),
          _full((16, 64)),
          _full((16, 64)),
          _full((16, 64)),
          _full((1, 64)),
          _full((1, 64)),
      ],
      out_specs=[_rows((2, RB, 32)), _rows((RB, 8)), _rows((8, 128))],
      out_shape=[
          jax.ShapeDtypeStruct((2, NN, 32), jnp.float32),
          jax.ShapeDtypeStruct((NN, 8), jnp.float32),
          jax.ShapeDtypeStruct((NN // RB * 8, 128), jnp.float32),
      ],
  )(s1, f1, wl1, wr1, wres1, b1f, bres1)


def _scales(stats):
  def body(st_ref, out_ref):
    habs = jnp.max(st_ref[:, :64], axis=0, keepdims=True)
    dmax = jnp.max(st_ref[:, 64:])
    bias = jnp.floor(65535.0 / (2.0 * jnp.maximum(dmax, 1.0)))
    s = (bias - 1.0) / jnp.maximum(habs, 1e-20)
    out_ref[0:1] = s
    out_ref[1:2] = 1.0 / s
    out_ref[2:3] = jnp.full((1, 64), bias)
    out_ref[3:4] = jnp.zeros((1, 64), jnp.float32)

  return pl.pallas_call(
      body,
      grid=(1,),
      in_specs=[_full((NN // RB * 8, 128))],
      out_specs=_full((4, 64)),
      out_shape=jax.ShapeDtypeStruct((4, 64), jnp.float32),
  )(stats)


def _pack(hb, scales):
  def body(h_ref, sc_ref, out_ref):
    h = jnp.concatenate([h_ref[0], h_ref[1]], axis=1)
    q = jnp.rint(h * sc_ref[0:1]).astype(jnp.int32)
    bi = sc_ref[2, 0].astype(jnp.int32)
    out_ref[...] = (q[:, :32] + bi) * 65536 + (q[:, 32:] + bi)

  return pl.pallas_call(
      body,
      grid=(NN // RB,),
      in_specs=[_rows((2, RB, 32)), _full((4, 64))],
      out_specs=_rows((RB, 32)),
      out_shape=jax.ShapeDtypeStruct((NN, 32), jnp.int32),
  )(hb, scales)


def _unpack(p0, p1, cnt, scl_ref):
  sixteen = jnp.full(p0.shape, 16, jnp.int32)
  mask = jnp.full(p0.shape, 0xFFFF, jnp.int32)
  lo = jnp.bitwise_and(p0, mask) + jnp.bitwise_and(p1, mask)
  hi = (lax.shift_right_logical(p0, sixteen)
        + lax.shift_right_logical(p1, sixteen))
  hif = hi.astype(jnp.float32)
  lof = lo.astype(jnp.float32)
  return (jnp.concatenate([hif, lof], axis=1) - cnt * scl_ref[2, 0]
          ) * scl_ref[1:2]


def _dense2(s2, h1b, rcnt, scl1, wl2, wr2, wres2, wl3, b2f, bres2):
  def body(s2_ref, h1_ref, rc_ref, scl_ref, wl2_ref, wr2_ref, wres2_ref,
           wl3_ref, b2_ref, bres2_ref, h2_ref, g3_ref, st_ref):
    rc = rc_ref[:, 0:1]
    cnt = rc_ref[:, 4:5]
    mean = _unpack(s2_ref[0, 0], s2_ref[1, 0], cnt, scl_ref) * rc
    h1 = jnp.concatenate([h1_ref[b] for b in range(2)], axis=1)
    h = (jnp.dot(mean, wl2_ref[...], preferred_element_type=jnp.float32)
         + jnp.dot(h1, wr2_ref[...], preferred_element_type=jnp.float32)
         + b2_ref[...])
    h2 = jnp.maximum(h, 0.0) + jnp.dot(
        h1, wres2_ref[...], preferred_element_type=jnp.float32) + bres2_ref[...]
    h2_ref[...] = h2
    g3 = jnp.dot(h2, wl3_ref[...], preferred_element_type=jnp.float32)
    habs = jnp.max(jnp.abs(g3), axis=0, keepdims=True)
    dmax = jnp.full((1, 64), jnp.max(cnt))
    st_ref[...] = jnp.broadcast_to(
        jnp.concatenate([habs, dmax], axis=1), (8, 128))
    for b in range(2):
      g3_ref[b] = g3[:, b * 32:(b + 1) * 32]

  return pl.pallas_call(
      body,
      grid=(NN // RB,),
      in_specs=[
          _rows((2, 1, RB, 32)),
          _rows((2, RB, 32)),
          _rows((RB, 8)),
          _full((4, 64)),
          _full((64, 128)),
          _full((64, 128)),
          _full((64, 128)),
          _full((128, 64)),
          _full((1, 128)),
          _full((1, 128)),
      ],
      out_specs=[_rows((RB, 128)), _rows((2, RB, 32)), _rows((8, 128))],
      out_shape=[
          jax.ShapeDtypeStruct((NN, 128), jnp.float32),
          jax.ShapeDtypeStruct((2, NN, 32), jnp.float32),
          jax.ShapeDtypeStruct((NN // RB * 8, 128), jnp.float32),
      ],
  )(s2, h1b, rcnt, scl1, wl2, wr2, wres2, wl3, b2f, bres2)


def _dense3(s3, h2, rcnt, scl3, wr3, wres3, b3f, bres3, wc1, bc1, wc2p, bc2p,
            we1, bwe1, we2p, bwe2p):
  def body(s3_ref, h2_ref, rc_ref, scl_ref, wr3_ref, wres3_ref, b3_ref,
           bres3_ref, wc1_ref, bc1_ref, wc2_ref, bc2_ref, we1_ref, bwe1_ref,
           we2_ref, bwe2_ref, h3_ref, mule_ref, risk_ref):
    rc = rc_ref[:, 0:1]
    cnt = rc_ref[:, 4:5]
    mean = _unpack(s3_ref[0, 0], s3_ref[1, 0], cnt, scl_ref) * rc
    h2 = h2_ref[...]
    h = mean + jnp.dot(h2, wr3_ref[...],
                       preferred_element_type=jnp.float32) + b3_ref[...]
    h3 = jnp.maximum(h, 0.0) + jnp.dot(
        h2, wres3_ref[...], preferred_element_type=jnp.float32) + bres3_ref[...]
    h3_ref[...] = h3
    c1 = jnp.maximum(
        jnp.dot(h3, wc1_ref[...], preferred_element_type=jnp.float32)
        + bc1_ref[...], 0.0)
    logits = jnp.dot(c1, wc2_ref[...],
                     preferred_element_type=jnp.float32) + bc2_ref[...]
    mule_ref[...] = jax.nn.sigmoid(logits)
    e1 = jnp.maximum(
        jnp.dot(h3, we1_ref[...], preferred_element_type=jnp.float32)
        + bwe1_ref[...], 0.0)
    el = jnp.dot(e1, we2_ref[...],
                 preferred_element_type=jnp.float32) + bwe2_ref[...]
    el = el - jnp.max(el, axis=1, keepdims=True)
    ex = jnp.exp(el)
    risk_ref[...] = ex / jnp.sum(ex, axis=1, keepdims=True)

  return pl.pallas_call(
      body,
      grid=(NN // RB,),
      in_specs=[
          _rows((2, 1, RB, 32)),
          _rows((RB, 128)),
          _rows((RB, 8)),
          _full((4, 64)),
          _full((128, 64)),
          _full((128, 64)),
          _full((1, 64)),
          _full((1, 64)),
          _full((64, 32)),
          _full((1, 32)),
          _full((32, 8)),
          _full((1, 8)),
          _full((64, 16)),
          _full((1, 16)),
          _full((16, 8)),
          _full((1, 8)),
      ],
      out_specs=[_rows((RB, 64)), _rows((RB, 8)), _rows((RB, 8))],
      out_shape=[
          jax.ShapeDtypeStruct((NN, 64), jnp.float32),
          jax.ShapeDtypeStruct((NN, 8), jnp.float32),
          jax.ShapeDtypeStruct((NN, 8), jnp.float32),
      ],
  )(s3, h2, rcnt, scl3, wr3, wres3, b3f, bres3, wc1, bc1, wc2p, bc2p, we1,
    bwe1, we2p, bwe2p)


def kernel(x, edge_index, W_l1, W_r1, b1, W_l2, W_r2, b2, W_l3, W_r3, b3,
           gn1, bt1, gn2, bt2, gn3, bt3,
           Wres1, bres1, Wres2, bres2, Wres3, bres3,
           Wc1, bc1, Wc2, bc2, We1, bwe1, We2, bwe2):
  n, in_c = x.shape
  e = edge_index.shape[1]

  eps = 1e-5
  sc1 = gn1 / jnp.sqrt(1.0 + eps)
  sc2 = gn2 / jnp.sqrt(1.0 + eps)
  sc3 = gn3 / jnp.sqrt(1.0 + eps)

  def pad_rows(w, rows):
    return jnp.concatenate(
        [w, jnp.zeros((rows - w.shape[0], w.shape[1]), w.dtype)], axis=0)

  wl1 = pad_rows(W_l1 * sc1[None, :], 16)
  wr1 = pad_rows(W_r1 * sc1[None, :], 16)
  wres1 = pad_rows(Wres1, 16)
  b1f = (b1 * sc1 + bt1)[None, :]
  bres1f = bres1[None, :]

  wl2 = W_l2 * sc2[None, :]
  wr2 = W_r2 * sc2[None, :]
  b2f = (b2 * sc2 + bt2)[None, :]
  bres2f = bres2[None, :]

  wl3s = W_l3 * sc3[None, :]
  wr3 = W_r3 * sc3[None, :]
  b3f = (b3 * sc3 + bt3)[None, :]
  bres3f = bres3[None, :]

  wc2p = jnp.concatenate([Wc2, jnp.zeros((32, 7), jnp.float32)], axis=1)
  bc2p = jnp.concatenate([bc2, jnp.zeros((7,), jnp.float32)])[None, :]
  we2p = jnp.concatenate([We2, jnp.zeros((16, 3), jnp.float32)], axis=1)
  bwe2p = jnp.concatenate(
      [bwe2, jnp.full((3,), -1e30, jnp.float32)])[None, :]
  bc1f = bc1[None, :]
  bwe1f = bwe1[None, :]

  src = edge_index[0]
  dst = edge_index[1]
  npad = E_PAD - e
  pad_dst = NN + jax.lax.rem(jnp.arange(npad, dtype=jnp.int32),
                             jnp.int32(N_ACC - NN))
  src_p = jnp.concatenate(
      [src, jnp.zeros((npad,), jnp.int32)]).reshape(IDX_ROWS, 128)
  dst_p = jnp.concatenate([dst, pad_dst]).reshape(IDX_ROWS, 128)

  f1 = jnp.concatenate(
      [x, jnp.ones((n, 1), jnp.float32), jnp.zeros((n, 5), jnp.float32)],
      axis=1)

  s1 = _agg(1, 16, f1.reshape(1, n, 16), src_p, dst_p)
  h1b, rcnt, stats1 = _dense1(s1, f1, wl1, wr1, wres1, b1f, bres1f)
  scl1 = _scales(stats1)
  h1q = _pack(h1b, scl1).reshape(1, n, 32)
  s2 = _agg(1, 32, h1q, src_p, dst_p)
  h2, g3b, stats3 = _dense2(s2, h1b, rcnt, scl1, wl2, wr2, Wres2, wl3s, b2f,
                            bres2f)
  scl3 = _scales(stats3)
  g3q = _pack(g3b, scl3).reshape(1, n, 32)
  s3 = _agg(1, 32, g3q, src_p, dst_p)
  h3, mule, risk = _dense3(s3, h2, rcnt, scl3, wr3, Wres3, b3f, bres3f, Wc1,
                           bc1f, wc2p, bc2p, We1, bwe1f, we2p, bwe2p)

  return (mule[:, 0], h3, risk[:, :5])
